# Initial kernel scaffold; baseline (speedup 1.0000x reference)
import jax
import jax.numpy as jnp
from jax.experimental import pallas as pl


def kernel(x_nhwc, w1, b1, wb1, bb1, gamma, beta, wb2, bb2, ws, bs, w2, b2, wd, bd):
    raise NotImplementedError("write your pallas kernel here")


# speedup vs baseline: 1.5557x; 1.5557x over previous
import jax
import jax.numpy as jnp
from jax.experimental import pallas as pl
from jax.experimental.pallas import tpu as pltpu

_EPS = 1e-5
_IMGS = 4


def _fold3x3(w, W):
    shift = jnp.stack([jnp.eye(W, W, k=1 - dx, dtype=w.dtype) for dx in range(3)])
    band = jnp.einsum("dij,ydab->yiajb", shift, w)
    return band.reshape(3, W * w.shape[2], W * w.shape[3])


def _fold1x1(w, W):
    return jnp.kron(jnp.eye(W, dtype=w.dtype), w)


def _fold1x1_s2(w, W):
    pick = jnp.eye(W, dtype=w.dtype)[:, ::2]
    return jnp.einsum("ij,ab->iajb", pick, w).reshape(W * w.shape[0], (W // 2) * w.shape[1])


def _row(v, W):
    return jnp.tile(v.astype(jnp.float32), W)[None, :]


def _scatter_padded(pad_ref, imgs, H):
    S = H + 2
    zero = jnp.zeros((1, pad_ref.shape[1]), pad_ref.dtype)
    for i, img in enumerate(imgs):
        base = i * S
        pad_ref[base:base + 1, :] = zero
        pad_ref[base + 1:base + 1 + H, :] = img
        pad_ref[base + S - 1:base + S, :] = zero


def _band_conv(pad_ref, w_ref, M):
    acc = jnp.dot(pad_ref[0:M, :], w_ref[0], preferred_element_type=jnp.float32)
    acc += jnp.dot(pad_ref[1:M + 1, :], w_ref[1], preferred_element_type=jnp.float32)
    acc += jnp.dot(pad_ref[2:M + 2, :], w_ref[2], preferred_element_type=jnp.float32)
    return acc


def _phase1(x_ref, w1_ref, b1_ref, wb1_ref, bb1_ref, t_ref, st_ref, pa_ref, pb_ref):
    B, H = x_ref.shape[0], x_ref.shape[1]
    S, M = H + 2, B * (H + 2) - 2

    _scatter_padded(pa_ref, [x_ref[i] for i in range(B)], H)
    x1 = jnp.maximum(_band_conv(pa_ref, w1_ref, M) + b1_ref[...], 0.0)

    _scatter_padded(pb_ref, [x1[i * S:i * S + H].astype(pb_ref.dtype) for i in range(B)], H)
    t = _band_conv(pb_ref, wb1_ref, M) + bb1_ref[...]

    s1 = jnp.zeros((1, t.shape[1]), jnp.float32)
    s2 = jnp.zeros((1, t.shape[1]), jnp.float32)
    for i in range(B):
        ti = t[i * S:i * S + H]
        t_ref[i] = ti.astype(t_ref.dtype)
        s1 += jnp.sum(ti, axis=0, keepdims=True)
        s2 += jnp.sum(ti * ti, axis=0, keepdims=True)
    st_ref[0:1, :] = s1
    st_ref[1:2, :] = s2


def _phase2(t_ref, x_ref, sc_ref, sh_ref, wb2_ref, bb2_ref, ws_ref, bs_ref,
            w2_ref, b2_ref, sel_ref, wd_ref, bd_ref, out_ref, dn_ref, pa_ref):
    B, H = t_ref.shape[0], t_ref.shape[1]
    S, M = H + 2, B * (H + 2) - 2
    Hh = H // 2

    tb = jnp.maximum(t_ref[...].astype(jnp.float32) * sc_ref[...] + sh_ref[...], 0.0)
    _scatter_padded(pa_ref, [tb[i].astype(pa_ref.dtype) for i in range(B)], H)
    acc = _band_conv(pa_ref, wb2_ref, M) + bb2_ref[...]

    xs = jnp.dot(x_ref[...].reshape(B * H, x_ref.shape[2]), ws_ref[...],
                 preferred_element_type=jnp.float32) + bs_ref[...]

    x3 = [acc[i * S:i * S + H] + xs[i * H:(i + 1) * H] for i in range(B)]
    _scatter_padded(pa_ref, [v.astype(pa_ref.dtype) for v in x3], H)
    out = jnp.maximum(_band_conv(pa_ref, w2_ref, M) + b2_ref[...], 0.0)

    dec = []
    for i in range(B):
        oi = out[i * S:i * S + H]
        out_ref[i] = oi
        dec.append(jnp.dot(sel_ref[...], oi.astype(sel_ref.dtype),
                           preferred_element_type=jnp.float32).astype(wd_ref.dtype))
    dn = jnp.dot(jnp.concatenate(dec, axis=0), wd_ref[...],
                 preferred_element_type=jnp.float32) + bd_ref[...]
    for i in range(B):
        dn_ref[i] = dn[i * Hh:(i + 1) * Hh]


def kernel(x_nhwc, w1, b1, wb1, bb1, gamma, beta, wb2, bb2, ws, bs, w2, b2, wd, bd):
    N, H, W, Cin = x_nhwc.shape
    Cout = w1.shape[-1]
    f32, bf16 = jnp.float32, jnp.bfloat16
    Wci, Wco, Wcd = W * Cin, W * Cout, (W // 2) * Cout

    B = _IMGS
    while N % B:
        B //= 2
    G = N // B

    x2d = x_nhwc.reshape(N, H, Wci).astype(bf16)

    w1b = _fold3x3(w1, W).astype(bf16)
    wb1b = _fold3x3(wb1, W).astype(bf16)
    wb2b = _fold3x3(wb2, W).astype(bf16)
    w2b = _fold3x3(w2, W).astype(bf16)
    wsb = _fold1x1(ws, W).astype(bf16)
    wdb = _fold1x1_s2(wd, W).astype(bf16)
    selb = jnp.eye(H, dtype=bf16)[::2, :]
    b1r, bb1r = _row(b1, W), _row(bb1, W)
    bb2r, bsr = _row(bb2, W), _row(bs, W)
    b2r, bdr = _row(b2, W), _row(bd, W // 2)

    par = pltpu.CompilerParams(dimension_semantics=("parallel",))
    rep2 = lambda g: (0, 0)
    rep3 = lambda g: (0, 0, 0)
    blk = lambda g: (g, 0, 0)

    t2d, stats = pl.pallas_call(
        _phase1,
        out_shape=(jax.ShapeDtypeStruct((N, H, Wco), bf16),
                   jax.ShapeDtypeStruct((G, 2, Wco), f32)),
        grid=(G,),
        in_specs=[
            pl.BlockSpec((B, H, Wci), blk),
            pl.BlockSpec((3, Wci, Wco), rep3),
            pl.BlockSpec((1, Wco), rep2),
            pl.BlockSpec((3, Wco, Wco), rep3),
            pl.BlockSpec((1, Wco), rep2),
        ],
        out_specs=(pl.BlockSpec((B, H, Wco), blk),
                   pl.BlockSpec((pl.Squeezed(), 2, Wco), blk)),
        scratch_shapes=[pltpu.VMEM((B * (H + 2), Wci), bf16),
                        pltpu.VMEM((B * (H + 2), Wco), bf16)],
        compiler_params=par,
    )(x2d, w1b, b1r, wb1b, bb1r)

    count = N * H * W
    s = stats.sum(axis=0).reshape(2, W, Cout).sum(axis=1)
    mean = s[0] / count
    var = s[1] / count - mean * mean
    scale = gamma / jnp.sqrt(var + _EPS)
    shift = beta - mean * scale
    scr, shr = _row(scale, W), _row(shift, W)

    out2d, down2d = pl.pallas_call(
        _phase2,
        out_shape=(jax.ShapeDtypeStruct((N, H, Wco), f32),
                   jax.ShapeDtypeStruct((N, H // 2, Wcd), f32)),
        grid=(G,),
        in_specs=[
            pl.BlockSpec((B, H, Wco), blk),
            pl.BlockSpec((B, H, Wci), blk),
            pl.BlockSpec((1, Wco), rep2),
            pl.BlockSpec((1, Wco), rep2),
            pl.BlockSpec((3, Wco, Wco), rep3),
            pl.BlockSpec((1, Wco), rep2),
            pl.BlockSpec((Wci, Wco), rep2),
            pl.BlockSpec((1, Wco), rep2),
            pl.BlockSpec((3, Wco, Wco), rep3),
            pl.BlockSpec((1, Wco), rep2),
            pl.BlockSpec((H // 2, H), rep2),
            pl.BlockSpec((Wco, Wcd), rep2),
            pl.BlockSpec((1, Wcd), rep2),
        ],
        out_specs=(pl.BlockSpec((B, H, Wco), blk),
                   pl.BlockSpec((B, H // 2, Wcd), blk)),
        scratch_shapes=[pltpu.VMEM((B * (H + 2), Wco), bf16)],
        compiler_params=par,
    )(t2d, x2d, scr, shr, wb2b, bb2r, wsb, bsr, w2b, b2r, selb, wdb, bdr)

    out = out2d.reshape(N, H, W, Cout)
    down = down2d.reshape(N, H // 2, W // 2, Cout)
    return out, down
